# baseline (speedup 1.0000x reference)
import functools

import jax
import jax.numpy as jnp
from jax import lax
from jax.experimental import pallas as pl
from jax.experimental.pallas import tpu as pltpu
from jax.experimental.pallas import tpu_sc as plsc


def _sc_mean(emb, idx3, B, E, L, NW, BPW, NCHUNK):
    IPW = BPW * L

    mesh = plsc.VectorSubcoreMesh(core_axis_name="c", subcore_axis_name="s")

    @functools.partial(
        pl.kernel,
        out_type=jax.ShapeDtypeStruct((B, E), jnp.float32),
        mesh=mesh,
        scratch_types=[
            pltpu.VMEM((NCHUNK, 128), jnp.int32),
            pltpu.VMEM((IPW, E), jnp.float32),
            pltpu.VMEM((BPW, E), jnp.float32),
            pltpu.SemaphoreType.DMA,
        ],
        compiler_params=pltpu.CompilerParams(use_tc_tiling_on_sc=False),
    )
    def run(emb_hbm, idx_hbm, out_hbm, idx_v, rows_v, acc_v, sem):
        wid = lax.axis_index("s") * 2 + lax.axis_index("c")
        pltpu.sync_copy(idx_hbm.at[wid], idx_v)
        copies = [
            pltpu.async_copy(emb_hbm.at[idx_v.at[j]],
                             rows_v.at[pl.ds(j * 128, 128)], sem)
            for j in range(NCHUNK)
        ]
        for c in copies:
            c.wait()

        inv_l = 1.0 / L

        def outer(bi, carry):
            def inner(j, acc):
                a0, a1 = acc
                r = bi * L + j
                a0 = a0 + rows_v[r, pl.ds(0, 16)]
                a1 = a1 + rows_v[r, pl.ds(16, 16)]
                return (a0, a1)

            z = jnp.zeros((16,), jnp.float32)
            a0, a1 = lax.fori_loop(0, L, inner, (z, z))
            acc_v[bi, pl.ds(0, 16)] = a0 * inv_l
            acc_v[bi, pl.ds(16, 16)] = a1 * inv_l
            return carry

        lax.fori_loop(0, BPW, outer, 0)
        pltpu.sync_copy(acc_v, out_hbm.at[pl.ds(wid * BPW, BPW)])

    return run(emb, idx3)


def _project(means, W, b2, B, E, V, VB):
    def body(m_ref, w_ref, b_ref, o_ref):
        o_ref[...] = lax.dot_general(
            m_ref[...], w_ref[...],
            (((1,), (1,)), ((), ())),
            preferred_element_type=jnp.float32,
        ) + b_ref[...]

    return pl.pallas_call(
        body,
        grid=(pl.cdiv(V, VB),),
        in_specs=[
            pl.BlockSpec((B, E), lambda j: (0, 0)),
            pl.BlockSpec((VB, E), lambda j: (j, 0)),
            pl.BlockSpec((1, VB), lambda j: (0, j)),
        ],
        out_specs=pl.BlockSpec((B, VB), lambda j: (0, j)),
        out_shape=jax.ShapeDtypeStruct((B, V), jnp.float32),
    )(means, W, b2)


def kernel(inputs, emb, W, b):
    B, S = inputs.shape
    V, E = emb.shape
    L = S - 1
    NW = 32
    BPW = B // NW
    NCHUNK = (BPW * L) // 128

    ctx = inputs[:, :L].astype(jnp.int32)
    idx3 = ctx.reshape(NW, NCHUNK, 128)
    means = _sc_mean(emb, idx3, B, E, L, NW, BPW, NCHUNK)
    return _project(means, W, b.reshape(1, V), B, E, V, VB=1024)


# speedup vs baseline: 1.0329x; 1.0329x over previous
import functools

import jax
import jax.numpy as jnp
from jax import lax
from jax.experimental import pallas as pl
from jax.experimental.pallas import tpu as pltpu
from jax.experimental.pallas import tpu_sc as plsc

_LANES = 128


def _sc_mean(emb128, idx_flat, B, L, NW, BPW, NCHUNK):
    IPW = BPW * L

    mesh = plsc.VectorSubcoreMesh(core_axis_name="c", subcore_axis_name="s")

    @functools.partial(
        pl.kernel,
        out_type=jax.ShapeDtypeStruct((B, _LANES), jnp.float32),
        mesh=mesh,
        scratch_types=[
            pltpu.VMEM((IPW,), jnp.int32),
            pltpu.VMEM((IPW, _LANES), jnp.float32),
            pltpu.VMEM((BPW, _LANES), jnp.float32),
            pltpu.SemaphoreType.DMA,
        ],
    )
    def run(emb_hbm, idx_hbm, out_hbm, idx_v, rows_v, acc_v, sem):
        wid = lax.axis_index("s") * 2 + lax.axis_index("c")
        pltpu.sync_copy(idx_hbm.at[pl.ds(wid * IPW, IPW)], idx_v)
        copies = [
            pltpu.async_copy(emb_hbm.at[idx_v.at[pl.ds(j * 128, 128)]],
                             rows_v.at[pl.ds(j * 128, 128)], sem)
            for j in range(NCHUNK)
        ]
        for c in copies:
            c.wait()

        inv_l = 1.0 / L

        def outer(bi, carry):
            def inner(j, acc):
                a0, a1 = acc
                r = bi * L + j
                a0 = a0 + rows_v[r, pl.ds(0, 16)]
                a1 = a1 + rows_v[r, pl.ds(16, 16)]
                return (a0, a1)

            z = jnp.zeros((16,), jnp.float32)
            a0, a1 = lax.fori_loop(0, L, inner, (z, z))
            acc_v[bi, pl.ds(0, 16)] = a0 * inv_l
            acc_v[bi, pl.ds(16, 16)] = a1 * inv_l
            return carry

        lax.fori_loop(0, BPW, outer, 0)
        pltpu.sync_copy(acc_v, out_hbm.at[pl.ds(wid * BPW, BPW)])

    return run(emb128, idx_flat)


def _project(means128, W, b2, B, E, V, VB):
    def body(m_ref, w_ref, b_ref, o_ref):
        o_ref[...] = lax.dot_general(
            m_ref[:, :E], w_ref[...],
            (((1,), (1,)), ((), ())),
            preferred_element_type=jnp.float32,
        ) + b_ref[...]

    return pl.pallas_call(
        body,
        grid=(pl.cdiv(V, VB),),
        in_specs=[
            pl.BlockSpec((B, _LANES), lambda j: (0, 0)),
            pl.BlockSpec((VB, E), lambda j: (j, 0)),
            pl.BlockSpec((1, VB), lambda j: (0, j)),
        ],
        out_specs=pl.BlockSpec((B, VB), lambda j: (0, j)),
        out_shape=jax.ShapeDtypeStruct((B, V), jnp.float32),
    )(means128, W, b2)


def kernel(inputs, emb, W, b):
    B, S = inputs.shape
    V, E = emb.shape
    L = S - 1
    NW = 32
    BPW = B // NW
    NCHUNK = (BPW * L) // 128

    emb128 = jnp.pad(emb, ((0, 0), (0, _LANES - E)))
    idx_flat = inputs[:, :L].astype(jnp.int32).reshape(B * L)
    means128 = _sc_mean(emb128, idx_flat, B, L, NW, BPW, NCHUNK)
    return _project(means128, W, b.reshape(1, V), B, E, V, VB=2048)


# speedup vs baseline: 2.7236x; 2.6370x over previous
import functools

import jax
import jax.numpy as jnp
from jax import lax
from jax.experimental import pallas as pl
from jax.experimental.pallas import tpu as pltpu
from jax.experimental.pallas import tpu_sc as plsc

_LANES = 128
_K = 48


def _sc_mean(emb128, inT, B, L, NW, BPW):
    IPW = BPW * L

    mesh = plsc.VectorSubcoreMesh(core_axis_name="c", subcore_axis_name="s")

    @functools.partial(
        pl.kernel,
        out_type=jax.ShapeDtypeStruct((B, _K), jnp.float32),
        mesh=mesh,
        scratch_types=[
            pltpu.VMEM((L * B,), jnp.int32),
            pltpu.VMEM((IPW, _LANES), jnp.float32),
            pltpu.VMEM((BPW, _K), jnp.float32),
            pltpu.SemaphoreType.DMA,
        ],
    )
    def run(emb_hbm, in_hbm, out_hbm, iv, rows_v, acc_v, sem):
        wid = lax.axis_index("s") * 2 + lax.axis_index("c")
        base = wid * BPW
        for j in range(L):
            pltpu.sync_copy(in_hbm.at[j], iv.at[pl.ds(j * B, B)])
        copies = [
            pltpu.async_copy(emb_hbm.at[iv.at[pl.ds(j * B + base, BPW)]],
                             rows_v.at[pl.ds(j * BPW, BPW)], sem)
            for j in range(L)
        ]
        for c in copies:
            c.wait()

        inv_l = 1.0 / L

        def outer(bi, carry):
            def inner(j, acc):
                a0, a1 = acc
                r = j * BPW + bi
                a0 = a0 + rows_v[r, pl.ds(0, 16)]
                a1 = a1 + rows_v[r, pl.ds(16, 16)]
                return (a0, a1)

            z = jnp.zeros((16,), jnp.float32)
            a0, a1 = lax.fori_loop(0, L, inner, (z, z))
            acc_v[bi, pl.ds(0, 16)] = a0 * inv_l
            acc_v[bi, pl.ds(16, 16)] = a1 * inv_l
            return carry

        lax.fori_loop(0, BPW, outer, 0)
        pltpu.sync_copy(acc_v, out_hbm.at[pl.ds(base, BPW)])

    return run(emb128, inT)


def _project_t(wtb, means, E, V, B, VB):
    def body(w_ref, m_ref, o_ref):
        m = m_ref[...]
        lane = lax.broadcasted_iota(jnp.int32, (B, _K), 1)
        m = jnp.where(lane == E, 1.0, jnp.where(lane > E, 0.0, m))
        o_ref[...] = lax.dot_general(
            w_ref[...], m,
            (((0,), (1,)), ((), ())),
            preferred_element_type=jnp.float32,
        )

    return pl.pallas_call(
        body,
        grid=(pl.cdiv(V, VB),),
        in_specs=[
            pl.BlockSpec((_K, VB), lambda j: (0, j)),
            pl.BlockSpec((B, _K), lambda j: (0, 0)),
        ],
        out_specs=pl.BlockSpec((VB, B), lambda j: (j, 0)),
        out_shape=jax.ShapeDtypeStruct((V, B), jnp.float32),
    )(wtb, means)


def kernel(inputs, emb, W, b):
    B, S = inputs.shape
    V, E = emb.shape
    L = S - 1
    NW = 32
    BPW = B // NW

    emb128 = jnp.pad(emb, ((0, 0), (0, _LANES - E)))
    inT = inputs.T.astype(jnp.int32)
    means = _sc_mean(emb128, inT, B, L, NW, BPW)
    wtb = jnp.concatenate(
        [W.T, b[None, :], jnp.zeros((_K - E - 1, V), jnp.float32)], axis=0
    )
    ot = _project_t(wtb, means, E, V, B, VB=2048)
    return ot.T


# speedup vs baseline: 2.8552x; 1.0483x over previous
import functools

import jax
import jax.numpy as jnp
from jax import lax
from jax.experimental import pallas as pl
from jax.experimental.pallas import tpu as pltpu
from jax.experimental.pallas import tpu_sc as plsc

_LANES = 128
_K = 48


def _sc_mean(emb128, inT, B, L, NW, BPW):
    IPW = BPW * L

    mesh = plsc.VectorSubcoreMesh(core_axis_name="c", subcore_axis_name="s")

    @functools.partial(
        pl.kernel,
        out_type=jax.ShapeDtypeStruct((B, _K), jnp.float32),
        mesh=mesh,
        scratch_types=[
            pltpu.VMEM((L * B,), jnp.int32),
            pltpu.VMEM((IPW, _LANES), jnp.float32),
            pltpu.VMEM((BPW, _K), jnp.float32),
            pltpu.SemaphoreType.DMA,
            pltpu.SemaphoreType.DMA,
        ],
    )
    def run(emb_hbm, in_hbm, out_hbm, iv, rows_v, acc_v, sem, sem2):
        wid = lax.axis_index("s") * 2 + lax.axis_index("c")
        base = wid * BPW
        stages = [
            pltpu.async_copy(in_hbm.at[j], iv.at[pl.ds(j * B, B)], sem2)
            for j in range(L)
        ]
        for s in stages:
            s.wait()
        copies = [
            pltpu.async_copy(emb_hbm.at[iv.at[pl.ds(j * B + base, BPW)]],
                             rows_v.at[pl.ds(j * BPW, BPW)], sem)
            for j in range(L)
        ]
        for c in copies:
            c.wait()

        inv_l = 1.0 / L

        def outer(bi, carry):
            def inner(j, acc):
                a0, a1 = acc
                r = j * BPW + bi
                a0 = a0 + rows_v[r, pl.ds(0, 16)]
                a1 = a1 + rows_v[r, pl.ds(16, 16)]
                return (a0, a1)

            z = jnp.zeros((16,), jnp.float32)
            a0, a1 = lax.fori_loop(0, L, inner, (z, z))
            acc_v[bi, pl.ds(0, 16)] = a0 * inv_l
            acc_v[bi, pl.ds(16, 16)] = a1 * inv_l
            return carry

        lax.fori_loop(0, BPW, outer, 0)
        pltpu.sync_copy(acc_v, out_hbm.at[pl.ds(base, BPW)])

    return run(emb128, inT)


def _project_t(wtb, means, E, V, B, VB):
    def body(w_ref, m_ref, o_ref):
        m = m_ref[...]
        lane = lax.broadcasted_iota(jnp.int32, (B, _K), 1)
        m = jnp.where(lane == E, 1.0, jnp.where(lane > E, 0.0, m))
        o_ref[...] = lax.dot_general(
            w_ref[...], m,
            (((0,), (1,)), ((), ())),
            preferred_element_type=jnp.float32,
        )

    return pl.pallas_call(
        body,
        grid=(pl.cdiv(V, VB),),
        in_specs=[
            pl.BlockSpec((_K, VB), lambda j: (0, j)),
            pl.BlockSpec((B, _K), lambda j: (0, 0)),
        ],
        out_specs=pl.BlockSpec((VB, B), lambda j: (j, 0)),
        out_shape=jax.ShapeDtypeStruct((V, B), jnp.float32),
    )(wtb, means)


def kernel(inputs, emb, W, b):
    B, S = inputs.shape
    V, E = emb.shape
    L = S - 1
    NW = 32
    BPW = B // NW

    emb128 = jnp.pad(emb, ((0, 0), (0, _LANES - E)))
    inT = inputs.T.astype(jnp.int32)
    means = _sc_mean(emb128, inT, B, L, NW, BPW)
    wtb = jnp.concatenate(
        [W.T, b[None, :], jnp.zeros((_K - E - 1, V), jnp.float32)], axis=0
    )
    ot = _project_t(wtb, means, E, V, B, VB=2048)
    return ot.T
